# baseline (speedup 1.0000x reference)
import functools

import jax
import jax.numpy as jnp
from jax import lax
from jax.experimental import pallas as pl
from jax.experimental.pallas import tpu as pltpu
from jax.experimental.pallas import tpu_sc as plsc

_NUM_CORES = 2
_NUM_SUBCORES = 16
_NUM_WORKERS = _NUM_CORES * _NUM_SUBCORES
_LANES = 16


@functools.cache
def _build_gather(rows: int, cols: int, vocab: int):
    stripe = cols // _NUM_WORKERS
    pad = lambda x, m: -(-x // m) * m
    rpc = rows
    while rpc > 1 and (
        rows % rpc
        or pad(vocab, 128) + 4 * pad(rpc, 8) * pad(stripe, 128) > 126976
    ):
        rpc -= 1
    n_chunks = rows // rpc

    mesh = plsc.VectorSubcoreMesh(core_axis_name="c", subcore_axis_name="s")

    pad8 = lambda x: -(-x // 8) * 8
    va = min(vocab, pad8(int(vocab * 0.36)))
    vb = vocab - va
    slice_w = pad8(-(-vb // _NUM_SUBCORES))
    last_w = vb - slice_w * (_NUM_SUBCORES - 1)

    @functools.partial(
        pl.kernel,
        out_type=jax.ShapeDtypeStruct((rows, cols), jnp.float32),
        mesh=mesh,
        compiler_params=pltpu.CompilerParams(needs_layout_passes=False),
        scratch_types=[
            pltpu.VMEM((vocab,), jnp.float32),
            pltpu.VMEM_SHARED((vb,), jnp.float32),
            pltpu.VMEM((2, rpc, stripe), jnp.int32),
            pltpu.VMEM((2, rpc, stripe), jnp.float32),
            pltpu.SemaphoreType.DMA,
            pltpu.SemaphoreType.DMA,
            pltpu.SemaphoreType.DMA,
            pltpu.SemaphoreType.DMA,
            pltpu.SemaphoreType.DMA,
            pltpu.SemaphoreType.DMA,
        ],
    )
    def gather_kernel(
        idx_hbm, table_hbm, out_hbm, table_v, table_sh, idx_v, out_v,
        sem_t, sem_t2, sem_i0, sem_i1, sem_o0, sem_o1,
    ):
        sem_i = (sem_i0, sem_i1)
        sem_o = (sem_o0, sem_o1)
        sid = lax.axis_index("s")
        wid = sid * _NUM_CORES + lax.axis_index("c")
        col0 = wid * stripe

        in_cps = [None, None]
        out_cps = [None, None]
        for c in range(min(2, n_chunks)):
            in_cps[c] = pltpu.async_copy(
                idx_hbm.at[pl.ds(c * rpc, rpc), pl.ds(col0, stripe)],
                idx_v.at[c],
                sem_i[c],
            )

        part_a_cp = pltpu.async_copy(
            table_hbm.at[pl.ds(0, va)], table_v.at[pl.ds(0, va)], sem_t2
        )

        for k in range(_NUM_SUBCORES):
            base = k * slice_w
            width = slice_w if k < _NUM_SUBCORES - 1 else last_w

            @pl.when(sid == k)
            def _(base=base, width=width):
                pltpu.async_copy(
                    table_hbm.at[pl.ds(va + base, width)],
                    table_v.at[pl.ds(va + base, width)],
                    sem_t,
                ).wait()
                pltpu.async_copy(
                    table_v.at[pl.ds(va + base, width)],
                    table_sh.at[pl.ds(base, width)],
                    sem_t,
                ).wait()

        plsc.subcore_barrier()
        pltpu.async_copy(table_sh, table_v.at[pl.ds(va, vb)], sem_t).wait()
        part_a_cp.wait()

        for c in range(n_chunks):
            buf = c % 2
            in_cps[buf].wait()
            if out_cps[buf] is not None:
                out_cps[buf].wait()

            def row_body(r):
                for o in range(0, stripe, _LANES):
                    ivec = idx_v[buf, r, pl.ds(o, _LANES)]
                    out_v[buf, r, pl.ds(o, _LANES)] = plsc.load_gather(
                        table_v, [ivec]
                    )

            plsc.parallel_loop(0, rpc, 1, unroll=2)(row_body)

            out_cps[buf] = pltpu.async_copy(
                out_v.at[buf],
                out_hbm.at[pl.ds(c * rpc, rpc), pl.ds(col0, stripe)],
                sem_o[buf],
            )
            if c + 2 < n_chunks:
                in_cps[buf] = pltpu.async_copy(
                    idx_hbm.at[pl.ds((c + 2) * rpc, rpc), pl.ds(col0, stripe)],
                    idx_v.at[buf],
                    sem_i[buf],
                )
        for cp in out_cps:
            if cp is not None:
                cp.wait()

    return gather_kernel


def kernel(tok_iter, vocab_table):
    b, h = tok_iter.shape
    out_t = _build_gather(h, b, vocab_table.shape[0])(
        tok_iter.T, vocab_table
    )
    return out_t.T


# speedup vs baseline: 1.0470x; 1.0470x over previous
import functools

import jax
import jax.numpy as jnp
from jax import lax
from jax.experimental import pallas as pl
from jax.experimental.pallas import tpu as pltpu
from jax.experimental.pallas import tpu_sc as plsc

_NUM_CORES = 2
_NUM_SUBCORES = 16
_NUM_WORKERS = _NUM_CORES * _NUM_SUBCORES
_LANES = 16


@functools.cache
def _build_gather(rows: int, cols: int, vocab: int):
    stripe = cols // _NUM_WORKERS
    pad = lambda x, m: -(-x // m) * m
    pad8 = lambda x: pad(x, 8)
    slice_w = pad8(-(-vocab // _NUM_SUBCORES))
    last_w = vocab - slice_w * (_NUM_SUBCORES - 1)
    rpc = rows
    while rpc > 1 and (
        rows % rpc
        or slice_w + 4 * pad8(rpc) * pad(stripe, 128) > 126976
    ):
        rpc -= 1
    n_chunks = rows // rpc

    mesh = plsc.VectorSubcoreMesh(core_axis_name="c", subcore_axis_name="s")

    @functools.partial(
        pl.kernel,
        out_type=jax.ShapeDtypeStruct((rows, cols), jnp.float32),
        mesh=mesh,
        compiler_params=pltpu.CompilerParams(needs_layout_passes=False),
        scratch_types=[
            pltpu.VMEM((slice_w,), jnp.float32),
            pltpu.VMEM_SHARED((vocab,), jnp.float32),
            pltpu.VMEM((2, rpc, stripe), jnp.int32),
            pltpu.VMEM((2, rpc, stripe), jnp.float32),
            pltpu.SemaphoreType.DMA,
            pltpu.SemaphoreType.DMA,
            pltpu.SemaphoreType.DMA,
            pltpu.SemaphoreType.DMA,
            pltpu.SemaphoreType.DMA,
            pltpu.SemaphoreType.DMA,
        ],
    )
    def gather_kernel(
        idx_hbm, table_hbm, out_hbm, bounce_v, table_sh, idx_v, out_v,
        sem_t, sem_g, sem_i0, sem_i1, sem_o0, sem_o1,
    ):
        sem_i = (sem_i0, sem_i1)
        sem_o = (sem_o0, sem_o1)
        sid = lax.axis_index("s")
        wid = sid * _NUM_CORES + lax.axis_index("c")
        col0 = wid * stripe

        in_cps = [None, None]
        out_cps = [None, None]
        for c in range(min(2, n_chunks)):
            in_cps[c] = pltpu.async_copy(
                idx_hbm.at[pl.ds(c * rpc, rpc), pl.ds(col0, stripe)],
                idx_v.at[c],
                sem_i[c],
            )

        for k in range(_NUM_SUBCORES):
            base = k * slice_w
            width = slice_w if k < _NUM_SUBCORES - 1 else last_w

            @pl.when(sid == k)
            def _(base=base, width=width):
                pltpu.async_copy(
                    table_hbm.at[pl.ds(base, width)],
                    bounce_v.at[pl.ds(0, width)],
                    sem_t,
                ).wait()
                pltpu.async_copy(
                    bounce_v.at[pl.ds(0, width)],
                    table_sh.at[pl.ds(base, width)],
                    sem_t,
                ).wait()

        plsc.subcore_barrier()

        for c in range(n_chunks):
            buf = c % 2
            in_cps[buf].wait()
            if out_cps[buf] is not None:
                out_cps[buf].wait()

            def issue_row(r, _):
                pltpu.async_copy(
                    table_sh.at[idx_v.at[buf, r]],
                    out_v.at[buf, r],
                    sem_g,
                )
                return 0

            lax.fori_loop(0, rpc, issue_row, 0)
            pltpu.make_async_copy(
                out_hbm.at[pl.ds(c * rpc, rpc), pl.ds(col0, stripe)],
                out_v.at[buf],
                sem_g,
            ).wait()

            out_cps[buf] = pltpu.async_copy(
                out_v.at[buf],
                out_hbm.at[pl.ds(c * rpc, rpc), pl.ds(col0, stripe)],
                sem_o[buf],
            )
            if c + 2 < n_chunks:
                in_cps[buf] = pltpu.async_copy(
                    idx_hbm.at[pl.ds((c + 2) * rpc, rpc), pl.ds(col0, stripe)],
                    idx_v.at[buf],
                    sem_i[buf],
                )
        for cp in out_cps:
            if cp is not None:
                cp.wait()

    return gather_kernel


def kernel(tok_iter, vocab_table):
    b, h = tok_iter.shape
    out_t = _build_gather(h, b, vocab_table.shape[0])(
        tok_iter.T, vocab_table
    )
    return out_t.T


# speedup vs baseline: 1.0487x; 1.0016x over previous
import functools

import jax
import jax.numpy as jnp
from jax import lax
from jax.experimental import pallas as pl
from jax.experimental.pallas import tpu as pltpu
from jax.experimental.pallas import tpu_sc as plsc

_NUM_CORES = 2
_NUM_SUBCORES = 16
_NUM_WORKERS = _NUM_CORES * _NUM_SUBCORES
_LANES = 16


@functools.cache
def _build_gather(rows: int, cols: int, vocab: int):
    stripe = cols // _NUM_WORKERS
    pad = lambda x, m: -(-x // m) * m
    pad8 = lambda x: pad(x, 8)
    slice_w = pad8(-(-vocab // _NUM_SUBCORES))
    last_w = vocab - slice_w * (_NUM_SUBCORES - 1)
    rpc = rows
    while rpc > 1 and (
        rows % rpc
        or rpc % 8
        or slice_w + 4 * pad8(rpc) * pad(stripe, 128) > 126976
    ):
        rpc -= 1
    n_chunks = rows // rpc

    mesh = plsc.VectorSubcoreMesh(core_axis_name="c", subcore_axis_name="s")

    @functools.partial(
        pl.kernel,
        out_type=jax.ShapeDtypeStruct((rows, cols), jnp.float32),
        mesh=mesh,
        compiler_params=pltpu.CompilerParams(needs_layout_passes=False),
        scratch_types=[
            pltpu.VMEM((slice_w,), jnp.float32),
            pltpu.VMEM_SHARED((vocab,), jnp.float32),
            pltpu.VMEM((2, rpc, stripe), jnp.int32),
            pltpu.VMEM((2, rpc, stripe), jnp.float32),
            pltpu.SemaphoreType.DMA,
            pltpu.SemaphoreType.DMA,
            pltpu.SemaphoreType.DMA,
            pltpu.SemaphoreType.DMA,
            pltpu.SemaphoreType.DMA,
            pltpu.SemaphoreType.DMA,
        ],
    )
    def gather_kernel(
        idx_hbm, table_hbm, out_hbm, bounce_v, table_sh, idx_v, out_v,
        sem_t, sem_g, sem_i0, sem_i1, sem_o0, sem_o1,
    ):
        sem_i = (sem_i0, sem_i1)
        sem_o = (sem_o0, sem_o1)
        sid = lax.axis_index("s")
        wid = sid * _NUM_CORES + lax.axis_index("c")
        col0 = wid * stripe

        in_cps = [None, None]
        out_cps = [None, None]
        for c in range(min(2, n_chunks)):
            in_cps[c] = pltpu.async_copy(
                idx_hbm.at[pl.ds(c * rpc, rpc), pl.ds(col0, stripe)],
                idx_v.at[c],
                sem_i[c],
            )

        for k in range(_NUM_SUBCORES):
            base = k * slice_w
            width = slice_w if k < _NUM_SUBCORES - 1 else last_w

            @pl.when(sid == k)
            def _(base=base, width=width):
                pltpu.async_copy(
                    table_hbm.at[pl.ds(base, width)],
                    bounce_v.at[pl.ds(0, width)],
                    sem_t,
                ).wait()
                pltpu.async_copy(
                    bounce_v.at[pl.ds(0, width)],
                    table_sh.at[pl.ds(base, width)],
                    sem_t,
                ).wait()

        plsc.subcore_barrier()

        for c in range(n_chunks):
            buf = c % 2
            in_cps[buf].wait()
            if out_cps[buf] is not None:
                out_cps[buf].wait()

            def issue_row(r, _):
                pltpu.async_copy(
                    table_sh.at[idx_v.at[buf, r]],
                    out_v.at[buf, r],
                    sem_g,
                )
                return 0

            lax.fori_loop(0, rpc, issue_row, 0)
            pltpu.make_async_copy(
                out_hbm.at[pl.ds(c * rpc, rpc), pl.ds(col0, stripe)],
                out_v.at[buf],
                sem_g,
            ).wait()

            out_cps[buf] = pltpu.async_copy(
                out_v.at[buf],
                out_hbm.at[pl.ds(c * rpc, rpc), pl.ds(col0, stripe)],
                sem_o[buf],
            )
            if c + 2 < n_chunks:
                in_cps[buf] = pltpu.async_copy(
                    idx_hbm.at[pl.ds((c + 2) * rpc, rpc), pl.ds(col0, stripe)],
                    idx_v.at[buf],
                    sem_i[buf],
                )
        for cp in out_cps:
            if cp is not None:
                cp.wait()

    return gather_kernel


def kernel(tok_iter, vocab_table):
    b, h = tok_iter.shape
    out_t = _build_gather(h, b, vocab_table.shape[0])(
        tok_iter.T, vocab_table
    )
    return out_t.T
